# Initial kernel scaffold; baseline (speedup 1.0000x reference)
import jax
import jax.numpy as jnp
from jax.experimental import pallas as pl


def kernel(xyz, params):
    raise NotImplementedError("write your pallas kernel here")


# speedup vs baseline: 11.3088x; 11.3088x over previous
import functools

import jax
import jax.numpy as jnp
from jax.experimental import pallas as pl

_INF = float('inf')


def _cdist_rows(a, xr):
    na = jnp.sum(a * a, axis=1, keepdims=True)
    nb = jnp.sum(xr * xr, axis=0, keepdims=True)
    cross = jax.lax.dot_general(
        a.astype(jnp.bfloat16), xr.astype(jnp.bfloat16),
        (((1,), (0,)), ((), ())), preferred_element_type=jnp.float32)
    return jnp.sqrt(jnp.maximum(na + nb - 2.0 * cross, 1e-12))


def _fps_body(x_ref, out_ref, *, npoint):
    x = x_ref[...]
    B, _, N = x.shape
    iota = jax.lax.broadcasted_iota(jnp.int32, (B, N), 1)

    def body(s, carry):
        distance, farthest = carry
        oh = (iota == farthest).astype(jnp.float32)
        cent = jnp.sum(x * oh[:, None, :], axis=2)
        out_ref[pl.ds(s, 1)] = cent[None]
        diff = x - cent[:, :, None]
        dist = jnp.sum(diff * diff, axis=1)
        distance = jnp.minimum(distance, dist)
        m = jnp.max(distance, axis=1, keepdims=True)
        farthest = jnp.min(jnp.where(distance == m, iota, N),
                           axis=1, keepdims=True)
        return distance, farthest

    init = (jnp.full((B, N), 1e10, jnp.float32), jnp.zeros((B, 1), jnp.int32))
    jax.lax.fori_loop(0, npoint, body, init)


def _fps(x_rows, npoint):
    B = x_rows.shape[0]
    out = pl.pallas_call(
        functools.partial(_fps_body, npoint=npoint),
        out_shape=jax.ShapeDtypeStruct((npoint, B, 3), jnp.float32),
    )(x_rows)
    return jnp.transpose(out, (1, 0, 2))


def _bq_body(xr_ref, cols_ref, nx_ref, out_ref, *, K, r2, nfeat):
    xr = xr_ref[0]
    cols = cols_ref[0]
    a = nx_ref[0]
    N = xr.shape[1]
    St = a.shape[0]
    d = _cdist_rows(a, xr)
    iota = jax.lax.broadcasted_iota(jnp.int32, (St, N), 1)
    inr = d < r2
    trip = jnp.minimum(jnp.max(jnp.sum(inr.astype(jnp.int32), axis=1)), K)
    mn = jnp.min(d, axis=1, keepdims=True)
    nearest = jnp.min(jnp.where(d == mn, iota, N), axis=1)
    if nfeat:
        sub = jnp.concatenate([a, jnp.zeros((St, nfeat), jnp.float32)], axis=1)
    else:
        sub = a
    ohn = (iota == nearest[:, None]).astype(jnp.float32)
    g0 = jax.lax.dot_general(ohn, cols, (((1,), (0,)), ((), ())),
                             precision=jax.lax.Precision.HIGHEST,
                             preferred_element_type=jnp.float32) - sub
    out_ref[0] = jnp.broadcast_to(g0[None], (K,) + g0.shape)
    dm0 = jnp.where(inr, d, _INF)

    def body(j, dm):
        mnj = jnp.min(dm, axis=1, keepdims=True)
        valid = mnj < _INF
        sel = jnp.min(jnp.where(dm == mnj, iota, N), axis=1)
        oh = (iota == sel[:, None]).astype(jnp.float32)
        g = jax.lax.dot_general(oh, cols, (((1,), (0,)), ((), ())),
                                precision=jax.lax.Precision.HIGHEST,
                                preferred_element_type=jnp.float32) - sub
        g = jnp.where(valid, g, g0)
        out_ref[0, pl.ds(j, 1)] = g[None]
        return jnp.where(oh > 0, _INF, dm)

    jax.lax.fori_loop(0, trip, body, dm0)


def _bq(x_rows, cols, nx, radius, K, St):
    B, _, N = x_rows.shape
    S = nx.shape[1]
    Call = cols.shape[2]
    out = pl.pallas_call(
        functools.partial(_bq_body, K=K, r2=radius * radius, nfeat=Call - 3),
        grid=(B, S // St),
        in_specs=[
            pl.BlockSpec((1, 3, N), lambda b, t: (b, 0, 0)),
            pl.BlockSpec((1, N, Call), lambda b, t: (b, 0, 0)),
            pl.BlockSpec((1, St, 3), lambda b, t: (b, t, 0)),
        ],
        out_specs=pl.BlockSpec((1, K, St, Call), lambda b, t: (b, 0, t, 0)),
        out_shape=jax.ShapeDtypeStruct((B, K, S, Call), jnp.float32),
    )(x_rows, cols, nx)
    return out


def _mlp_layer_body(*refs, inv_p, has_norm):
    if has_norm:
        x_ref, w_ref, b_ref, s_ref, g_ref, be_ref, z_ref, st_ref = refs
    else:
        x_ref, w_ref, b_ref, z_ref, st_ref = refs
    x = x_ref[...]
    if has_norm:
        s = s_ref[...]
        mean = s[0:1] * inv_p
        var = s[1:2] * inv_p - mean * mean
        scale = g_ref[...] * jax.lax.rsqrt(var + 1e-5)
        shift = be_ref[...] - mean * scale
        x = jnp.maximum(x * scale + shift, 0.0)
    z = jax.lax.dot_general(x.astype(jnp.bfloat16),
                            w_ref[...].astype(jnp.bfloat16),
                            (((1,), (1,)), ((), ())),
                            preferred_element_type=jnp.float32) + b_ref[...]
    z_ref[...] = z

    @pl.when(pl.program_id(0) == 0)
    def _():
        st_ref[...] = jnp.zeros_like(st_ref)

    st_ref[...] += jnp.concatenate(
        [jnp.sum(z, axis=0, keepdims=True),
         jnp.sum(z * z, axis=0, keepdims=True)], axis=0)


def _mlp_layer(x, w, b, norm, inv_p):
    P, Cin = x.shape
    Cout = w.shape[0]
    T = min(P, 8192)
    inputs = [x, w, b]
    in_specs = [
        pl.BlockSpec((T, Cin), lambda i: (i, 0)),
        pl.BlockSpec((Cout, Cin), lambda i: (0, 0)),
        pl.BlockSpec((1, Cout), lambda i: (0, 0)),
    ]
    if norm is not None:
        inputs += list(norm)
        in_specs += [
            pl.BlockSpec((2, Cin), lambda i: (0, 0)),
            pl.BlockSpec((1, Cin), lambda i: (0, 0)),
            pl.BlockSpec((1, Cin), lambda i: (0, 0)),
        ]
    return pl.pallas_call(
        functools.partial(_mlp_layer_body, inv_p=inv_p,
                          has_norm=norm is not None),
        grid=(P // T,),
        in_specs=in_specs,
        out_specs=[pl.BlockSpec((T, Cout), lambda i: (i, 0)),
                   pl.BlockSpec((2, Cout), lambda i: (0, 0))],
        out_shape=[jax.ShapeDtypeStruct((P, Cout), jnp.float32),
                   jax.ShapeDtypeStruct((2, Cout), jnp.float32)],
    )(*inputs)


def _norm_scale_shift(s, g, be, inv_p):
    mean = s[0:1] * inv_p
    var = s[1:2] * inv_p - mean * mean
    scale = g * jax.lax.rsqrt(var + 1e-5)
    return scale, be - mean * scale


def _pool_body(z_ref, s_ref, g_ref, be_ref, out_ref, *, inv_p):
    z = z_ref[0]
    scale, shift = _norm_scale_shift(s_ref[...], g_ref[...], be_ref[...], inv_p)
    y = jnp.maximum(z * scale[None] + shift[None], 0.0)
    out_ref[0] = jnp.max(y, axis=0)


def _pool(z, st, g, be, inv_p, Sp):
    B, K, S, C = z.shape
    return pl.pallas_call(
        functools.partial(_pool_body, inv_p=inv_p),
        grid=(B, S // Sp),
        in_specs=[
            pl.BlockSpec((1, K, Sp, C), lambda b, t: (b, 0, t, 0)),
            pl.BlockSpec((2, C), lambda b, t: (0, 0)),
            pl.BlockSpec((1, C), lambda b, t: (0, 0)),
            pl.BlockSpec((1, C), lambda b, t: (0, 0)),
        ],
        out_specs=pl.BlockSpec((1, Sp, C), lambda b, t: (b, t, 0)),
        out_shape=jax.ShapeDtypeStruct((B, S, C), jnp.float32),
    )(z, st, g, be)


def _normrelu_body(z_ref, s_ref, g_ref, be_ref, out_ref, *, inv_p):
    scale, shift = _norm_scale_shift(s_ref[...], g_ref[...], be_ref[...], inv_p)
    out_ref[...] = jnp.maximum(z_ref[...] * scale + shift, 0.0)


def _normrelu(z, st, g, be, inv_p):
    P, C = z.shape
    T = min(P, 8192)
    return pl.pallas_call(
        functools.partial(_normrelu_body, inv_p=inv_p),
        grid=(P // T,),
        in_specs=[
            pl.BlockSpec((T, C), lambda i: (i, 0)),
            pl.BlockSpec((2, C), lambda i: (0, 0)),
            pl.BlockSpec((1, C), lambda i: (0, 0)),
            pl.BlockSpec((1, C), lambda i: (0, 0)),
        ],
        out_specs=pl.BlockSpec((T, C), lambda i: (i, 0)),
        out_shape=jax.ShapeDtypeStruct((P, C), jnp.float32),
    )(z, st, g, be)


def _knn_body(a_ref, xr_ref, f_ref, out_ref):
    a = a_ref[0]
    xr = xr_ref[0]
    f2 = f_ref[0]
    N2 = xr.shape[1]
    d = _cdist_rows(a, xr)
    iota = jax.lax.broadcasted_iota(jnp.int32, d.shape, 1)
    acc = None
    wsum = None
    for _ in range(3):
        mn = jnp.min(d, axis=1, keepdims=True)
        sel = jnp.min(jnp.where(d == mn, iota, N2), axis=1)
        wj = 1.0 / (mn + 1e-8)
        oh = (iota == sel[:, None]).astype(jnp.float32)
        g = jax.lax.dot_general(oh, f2, (((1,), (0,)), ((), ())),
                                precision=jax.lax.Precision.HIGHEST,
                                preferred_element_type=jnp.float32)
        acc = wj * g if acc is None else acc + wj * g
        wsum = wj if wsum is None else wsum + wj
        d = jnp.where(oh > 0, _INF, d)
    out_ref[0] = acc / wsum


def _knn(xyz1_cols, xyz2_rows, feats2, T1):
    B, N1, _ = xyz1_cols.shape
    N2 = xyz2_rows.shape[2]
    C2 = feats2.shape[2]
    return pl.pallas_call(
        _knn_body,
        grid=(B, N1 // T1),
        in_specs=[
            pl.BlockSpec((1, T1, 3), lambda b, t: (b, t, 0)),
            pl.BlockSpec((1, 3, N2), lambda b, t: (b, 0, 0)),
            pl.BlockSpec((1, N2, C2), lambda b, t: (b, 0, 0)),
        ],
        out_specs=pl.BlockSpec((1, T1, C2), lambda b, t: (b, t, 0)),
        out_shape=jax.ShapeDtypeStruct((B, N1, C2), jnp.float32),
    )(xyz1_cols, xyz2_rows, feats2)


def _prep(layers):
    return [(w, b.reshape(1, -1), g.reshape(1, -1), be.reshape(1, -1))
            for (w, b, g, be) in layers]


def _mlp_chain(x, layers, inv_p):
    norm = None
    z = x
    for (w, b, g, be) in layers:
        z, st = _mlp_layer(z, w, b, norm, inv_p)
        norm = (st, g, be)
    return z, norm


def kernel(xyz, params):
    xyz = xyz.astype(jnp.float32)
    B, _, N = xyz.shape
    sa1 = _prep(params['sa1'])
    sa2 = _prep(params['sa2'])
    fp2 = _prep(params['fp2'])
    fp1 = _prep(params['fp1'])
    K = 32
    S1, S2 = 1024, 256

    xyz_cols = jnp.transpose(xyz, (0, 2, 1))

    nx1 = _fps(xyz, S1)
    g1 = _bq(xyz, xyz_cols, nx1, radius=0.1, K=K, St=128)
    p1 = B * K * S1
    z, norm = _mlp_chain(g1.reshape(p1, 3), sa1, 1.0 / p1)
    st, gg, bb = norm
    l1_pts = _pool(z.reshape(B, K, S1, 64), st, gg, bb, 1.0 / p1, Sp=512)

    nx1_rows = jnp.transpose(nx1, (0, 2, 1))
    nx2 = _fps(nx1_rows, S2)
    cols2 = jnp.concatenate([nx1, l1_pts], axis=2)
    g2 = _bq(nx1_rows, cols2, nx2, radius=0.2, K=K, St=256)
    p2 = B * K * S2
    z, norm = _mlp_chain(g2.reshape(p2, 67), sa2, 1.0 / p2)
    st, gg, bb = norm
    l2_pts = _pool(z.reshape(B, K, S2, 128), st, gg, bb, 1.0 / p2, Sp=256)

    nx2_rows = jnp.transpose(nx2, (0, 2, 1))
    interp2 = _knn(nx1, nx2_rows, l2_pts, T1=1024)
    pf2 = B * S1
    xf = jnp.concatenate([l1_pts, interp2], axis=2).reshape(pf2, 192)
    z, norm = _mlp_chain(xf, fp2, 1.0 / pf2)
    st, gg, bb = norm
    l1_new = _normrelu(z, st, gg, bb, 1.0 / pf2).reshape(B, S1, 128)

    interp1 = _knn(xyz_cols, nx1_rows, l1_new, T1=1024)
    pf1 = B * N
    z, norm = _mlp_chain(interp1.reshape(pf1, 128), fp1, 1.0 / pf1)
    st, gg, bb = norm
    out = _normrelu(z, st, gg, bb, 1.0 / pf1).reshape(B, N, 128)
    return jnp.transpose(out, (0, 2, 1))


# speedup vs baseline: 11.7677x; 1.0406x over previous
import functools

import jax
import jax.numpy as jnp
from jax.experimental import pallas as pl

_INF = float('inf')


def _cdist_rows(a, xr):
    na = jnp.sum(a * a, axis=1, keepdims=True)
    nb = jnp.sum(xr * xr, axis=0, keepdims=True)
    cross = jax.lax.dot_general(
        a.astype(jnp.bfloat16), xr.astype(jnp.bfloat16),
        (((1,), (0,)), ((), ())), preferred_element_type=jnp.float32)
    return jnp.sqrt(jnp.maximum(na + nb - 2.0 * cross, 1e-12))


def _fps_body(x_ref, out_ref, *, npoint):
    x = x_ref[...]
    B, _, N = x.shape
    iota = jax.lax.broadcasted_iota(jnp.int32, (B, N), 1)

    def body(s, carry):
        distance, farthest = carry
        oh = (iota == farthest).astype(jnp.float32)
        cent = jnp.sum(x * oh[:, None, :], axis=2)
        out_ref[pl.ds(s, 1)] = cent[None]
        diff = x - cent[:, :, None]
        dist = jnp.sum(diff * diff, axis=1)
        distance = jnp.minimum(distance, dist)
        m = jnp.max(distance, axis=1, keepdims=True)
        farthest = jnp.min(jnp.where(distance == m, iota, N),
                           axis=1, keepdims=True)
        return distance, farthest

    init = (jnp.full((B, N), 1e10, jnp.float32), jnp.zeros((B, 1), jnp.int32))
    jax.lax.fori_loop(0, npoint, body, init)


def _fps(x_rows, npoint):
    B = x_rows.shape[0]
    out = pl.pallas_call(
        functools.partial(_fps_body, npoint=npoint),
        out_shape=jax.ShapeDtypeStruct((npoint, B, 3), jnp.float32),
    )(x_rows)
    return jnp.transpose(out, (1, 0, 2))


def _bq_body(xr_ref, cols_ref, nx_ref, out_ref, *, K, r2, nfeat):
    xr = xr_ref[0]
    cols = cols_ref[0]
    a = nx_ref[0]
    N = xr.shape[1]
    St = a.shape[0]
    d = _cdist_rows(a, xr)
    iota = jax.lax.broadcasted_iota(jnp.int32, (St, N), 1)
    inr = d < r2
    trip = jnp.minimum(jnp.max(jnp.sum(inr.astype(jnp.int32), axis=1)), K)
    mn = jnp.min(d, axis=1, keepdims=True)
    nearest = jnp.min(jnp.where(d == mn, iota, N), axis=1)
    if nfeat:
        sub = jnp.concatenate([a, jnp.zeros((St, nfeat), jnp.float32)], axis=1)
    else:
        sub = a
    ohn = (iota == nearest[:, None]).astype(jnp.float32)
    g0 = jax.lax.dot_general(ohn, cols, (((1,), (0,)), ((), ())),
                             precision=jax.lax.Precision.HIGHEST,
                             preferred_element_type=jnp.float32) - sub
    out_ref[0] = jnp.broadcast_to(g0[None], (K,) + g0.shape)
    dm0 = jnp.where(inr, d, _INF)

    def body(j, dm):
        mnj = jnp.min(dm, axis=1, keepdims=True)
        valid = mnj < _INF
        sel = jnp.min(jnp.where(dm == mnj, iota, N), axis=1)
        oh = (iota == sel[:, None]).astype(jnp.float32)
        g = jax.lax.dot_general(oh, cols, (((1,), (0,)), ((), ())),
                                precision=jax.lax.Precision.HIGHEST,
                                preferred_element_type=jnp.float32) - sub
        g = jnp.where(valid, g, g0)
        out_ref[0, pl.ds(j, 1)] = g[None]
        return jnp.where(oh > 0, _INF, dm)

    jax.lax.fori_loop(0, trip, body, dm0)


def _bq(x_rows, cols, nx, radius, K, St):
    B, _, N = x_rows.shape
    S = nx.shape[1]
    Call = cols.shape[2]
    out = pl.pallas_call(
        functools.partial(_bq_body, K=K, r2=radius * radius, nfeat=Call - 3),
        grid=(B, S // St),
        in_specs=[
            pl.BlockSpec((1, 3, N), lambda b, t: (b, 0, 0)),
            pl.BlockSpec((1, N, Call), lambda b, t: (b, 0, 0)),
            pl.BlockSpec((1, St, 3), lambda b, t: (b, t, 0)),
        ],
        out_specs=pl.BlockSpec((1, K, St, Call), lambda b, t: (b, 0, t, 0)),
        out_shape=jax.ShapeDtypeStruct((B, K, S, Call), jnp.float32),
    )(x_rows, cols, nx)
    return out


def _mlp_layer_body(*refs, inv_p, has_norm):
    if has_norm:
        x_ref, w_ref, b_ref, s_ref, g_ref, be_ref, z_ref, st_ref = refs
    else:
        x_ref, w_ref, b_ref, z_ref, st_ref = refs
    x = x_ref[...]
    if has_norm:
        s = s_ref[...]
        mean = s[0:1] * inv_p
        var = s[1:2] * inv_p - mean * mean
        scale = g_ref[...] * jax.lax.rsqrt(var + 1e-5)
        shift = be_ref[...] - mean * scale
        x = jnp.maximum(x * scale + shift, 0.0)
    z = jax.lax.dot_general(x.astype(jnp.bfloat16),
                            w_ref[...].astype(jnp.bfloat16),
                            (((1,), (1,)), ((), ())),
                            preferred_element_type=jnp.float32) + b_ref[...]
    z_ref[...] = z

    @pl.when(pl.program_id(0) == 0)
    def _():
        st_ref[...] = jnp.zeros_like(st_ref)

    st_ref[...] += jnp.concatenate(
        [jnp.sum(z, axis=0, keepdims=True),
         jnp.sum(z * z, axis=0, keepdims=True)], axis=0)


def _mlp_layer(x, w, b, norm, inv_p):
    P, Cin = x.shape
    Cout = w.shape[0]
    T = min(P, 8192)
    inputs = [x, w, b]
    in_specs = [
        pl.BlockSpec((T, Cin), lambda i: (i, 0)),
        pl.BlockSpec((Cout, Cin), lambda i: (0, 0)),
        pl.BlockSpec((1, Cout), lambda i: (0, 0)),
    ]
    if norm is not None:
        inputs += list(norm)
        in_specs += [
            pl.BlockSpec((2, Cin), lambda i: (0, 0)),
            pl.BlockSpec((1, Cin), lambda i: (0, 0)),
            pl.BlockSpec((1, Cin), lambda i: (0, 0)),
        ]
    return pl.pallas_call(
        functools.partial(_mlp_layer_body, inv_p=inv_p,
                          has_norm=norm is not None),
        grid=(P // T,),
        in_specs=in_specs,
        out_specs=[pl.BlockSpec((T, Cout), lambda i: (i, 0)),
                   pl.BlockSpec((2, Cout), lambda i: (0, 0))],
        out_shape=[jax.ShapeDtypeStruct((P, Cout), jnp.float32),
                   jax.ShapeDtypeStruct((2, Cout), jnp.float32)],
    )(*inputs)


def _norm_scale_shift(s, g, be, inv_p):
    mean = s[0:1] * inv_p
    var = s[1:2] * inv_p - mean * mean
    scale = g * jax.lax.rsqrt(var + 1e-5)
    return scale, be - mean * scale


def _pool_body(z_ref, s_ref, g_ref, be_ref, out_ref, *, inv_p):
    z = z_ref[0]
    scale, shift = _norm_scale_shift(s_ref[...], g_ref[...], be_ref[...], inv_p)
    y = jnp.maximum(z * scale[None] + shift[None], 0.0)
    out_ref[0] = jnp.max(y, axis=0)


def _pool(z, st, g, be, inv_p, Sp):
    B, K, S, C = z.shape
    return pl.pallas_call(
        functools.partial(_pool_body, inv_p=inv_p),
        grid=(B, S // Sp),
        in_specs=[
            pl.BlockSpec((1, K, Sp, C), lambda b, t: (b, 0, t, 0)),
            pl.BlockSpec((2, C), lambda b, t: (0, 0)),
            pl.BlockSpec((1, C), lambda b, t: (0, 0)),
            pl.BlockSpec((1, C), lambda b, t: (0, 0)),
        ],
        out_specs=pl.BlockSpec((1, Sp, C), lambda b, t: (b, t, 0)),
        out_shape=jax.ShapeDtypeStruct((B, S, C), jnp.float32),
    )(z, st, g, be)


def _normrelu_body(z_ref, s_ref, g_ref, be_ref, out_ref, *, inv_p):
    scale, shift = _norm_scale_shift(s_ref[...], g_ref[...], be_ref[...], inv_p)
    out_ref[...] = jnp.maximum(z_ref[...] * scale + shift, 0.0)


def _normrelu(z, st, g, be, inv_p):
    P, C = z.shape
    T = min(P, 8192)
    return pl.pallas_call(
        functools.partial(_normrelu_body, inv_p=inv_p),
        grid=(P // T,),
        in_specs=[
            pl.BlockSpec((T, C), lambda i: (i, 0)),
            pl.BlockSpec((2, C), lambda i: (0, 0)),
            pl.BlockSpec((1, C), lambda i: (0, 0)),
            pl.BlockSpec((1, C), lambda i: (0, 0)),
        ],
        out_specs=pl.BlockSpec((T, C), lambda i: (i, 0)),
        out_shape=jax.ShapeDtypeStruct((P, C), jnp.float32),
    )(z, st, g, be)


def _knn_body(a_ref, xr_ref, f_ref, out_ref):
    a = a_ref[0]
    xr = xr_ref[0]
    f2 = f_ref[0]
    N2 = xr.shape[1]
    f_hi = f2.astype(jnp.bfloat16)
    r1 = f2 - f_hi.astype(jnp.float32)
    f_mid = r1.astype(jnp.bfloat16)
    f_lo = (r1 - f_mid.astype(jnp.float32)).astype(jnp.bfloat16)
    d = _cdist_rows(a, xr)
    iota = jax.lax.broadcasted_iota(jnp.int32, d.shape, 1)
    acc = None
    wsum = None
    for _ in range(3):
        mn = jnp.min(d, axis=1, keepdims=True)
        sel = jnp.min(jnp.where(d == mn, iota, N2), axis=1)
        wj = 1.0 / (mn + 1e-8)
        oh = (iota == sel[:, None]).astype(jnp.bfloat16)
        dims = (((1,), (0,)), ((), ()))
        g = (jax.lax.dot_general(oh, f_hi, dims,
                                 preferred_element_type=jnp.float32)
             + jax.lax.dot_general(oh, f_mid, dims,
                                   preferred_element_type=jnp.float32)
             + jax.lax.dot_general(oh, f_lo, dims,
                                   preferred_element_type=jnp.float32))
        acc = wj * g if acc is None else acc + wj * g
        wsum = wj if wsum is None else wsum + wj
        d = jnp.where(oh > 0, _INF, d)
    out_ref[0] = acc / wsum


def _knn(xyz1_cols, xyz2_rows, feats2, T1):
    B, N1, _ = xyz1_cols.shape
    N2 = xyz2_rows.shape[2]
    C2 = feats2.shape[2]
    return pl.pallas_call(
        _knn_body,
        grid=(B, N1 // T1),
        in_specs=[
            pl.BlockSpec((1, T1, 3), lambda b, t: (b, t, 0)),
            pl.BlockSpec((1, 3, N2), lambda b, t: (b, 0, 0)),
            pl.BlockSpec((1, N2, C2), lambda b, t: (b, 0, 0)),
        ],
        out_specs=pl.BlockSpec((1, T1, C2), lambda b, t: (b, t, 0)),
        out_shape=jax.ShapeDtypeStruct((B, N1, C2), jnp.float32),
    )(xyz1_cols, xyz2_rows, feats2)


def _prep(layers):
    return [(w, b.reshape(1, -1), g.reshape(1, -1), be.reshape(1, -1))
            for (w, b, g, be) in layers]


def _mlp_chain(x, layers, inv_p):
    norm = None
    z = x
    for (w, b, g, be) in layers:
        z, st = _mlp_layer(z, w, b, norm, inv_p)
        norm = (st, g, be)
    return z, norm


def kernel(xyz, params):
    xyz = xyz.astype(jnp.float32)
    B, _, N = xyz.shape
    sa1 = _prep(params['sa1'])
    sa2 = _prep(params['sa2'])
    fp2 = _prep(params['fp2'])
    fp1 = _prep(params['fp1'])
    K = 32
    S1, S2 = 1024, 256

    xyz_cols = jnp.transpose(xyz, (0, 2, 1))

    nx1 = _fps(xyz, S1)
    g1 = _bq(xyz, xyz_cols, nx1, radius=0.1, K=K, St=256)
    p1 = B * K * S1
    z, norm = _mlp_chain(g1.reshape(p1, 3), sa1, 1.0 / p1)
    st, gg, bb = norm
    l1_pts = _pool(z.reshape(B, K, S1, 64), st, gg, bb, 1.0 / p1, Sp=512)

    nx1_rows = jnp.transpose(nx1, (0, 2, 1))
    nx2 = _fps(nx1_rows, S2)
    cols2 = jnp.concatenate([nx1, l1_pts], axis=2)
    g2 = _bq(nx1_rows, cols2, nx2, radius=0.2, K=K, St=256)
    p2 = B * K * S2
    z, norm = _mlp_chain(g2.reshape(p2, 67), sa2, 1.0 / p2)
    st, gg, bb = norm
    l2_pts = _pool(z.reshape(B, K, S2, 128), st, gg, bb, 1.0 / p2, Sp=256)

    nx2_rows = jnp.transpose(nx2, (0, 2, 1))
    interp2 = _knn(nx1, nx2_rows, l2_pts, T1=1024)
    pf2 = B * S1
    xf = jnp.concatenate([l1_pts, interp2], axis=2).reshape(pf2, 192)
    z, norm = _mlp_chain(xf, fp2, 1.0 / pf2)
    st, gg, bb = norm
    l1_new = _normrelu(z, st, gg, bb, 1.0 / pf2).reshape(B, S1, 128)

    interp1 = _knn(xyz_cols, nx1_rows, l1_new, T1=1024)
    pf1 = B * N
    z, norm = _mlp_chain(interp1.reshape(pf1, 128), fp1, 1.0 / pf1)
    st, gg, bb = norm
    out = _normrelu(z, st, gg, bb, 1.0 / pf1).reshape(B, N, 128)
    return jnp.transpose(out, (0, 2, 1))
